# baseline (speedup 1.0000x reference)
import functools

import jax
import jax.numpy as jnp
from jax import lax
from jax.experimental import pallas as pl
from jax.experimental.pallas import tpu as pltpu
from jax.experimental.pallas import tpu_sc as plsc

N = 10000
NP = 10240
E = 320000
D = 128
L = 16
NC = 2
NS = 16
NW = NC * NS
EPW = E // NW
K = 80
NCHUNK = EPW // K
RPT = NP // NS
CPB = K
NCP = RPT // CPB

_mesh = plsc.VectorSubcoreMesh(
    core_axis_name="c", subcore_axis_name="s", num_cores=NC, num_subcores=NS
)


@functools.partial(
    pl.kernel,
    out_type=[
        jax.ShapeDtypeStruct((NC * NP, D), jnp.float32),
        jax.ShapeDtypeStruct((NW * NP,), jnp.float32),
    ],
    mesh=_mesh,
    compiler_params=pltpu.CompilerParams(needs_layout_passes=False),
    scratch_types=[
        pltpu.VMEM((K,), jnp.int32),
        pltpu.VMEM((K,), jnp.int32),
        pltpu.VMEM((K,), jnp.float32),
        pltpu.VMEM((K,), jnp.int32),
        pltpu.VMEM((K,), jnp.int32),
        pltpu.VMEM((K,), jnp.float32),
        pltpu.VMEM((K, D), jnp.float32),
        pltpu.VMEM((K, D), jnp.float32),
        pltpu.VMEM((NP,), jnp.float32),
        pltpu.VMEM_SHARED((NP, D), jnp.float32),
        pltpu.SemaphoreType.DMA,
        pltpu.SemaphoreType.DMA,
        pltpu.SemaphoreType.DMA,
        pltpu.SemaphoreType.DMA,
    ],
)
def _sc_accumulate(
    src_hbm, dst_hbm, w_hbm, x_hbm,
    acc_out, deg_out,
    src_v0, dst_v0, w_v0, src_v1, dst_v1, w_v1,
    rows_v0, rows_v1, deg_v, acc_sh, sem0, sem1, ssem0, ssem1,
):
    cid = lax.axis_index("c")
    sid = lax.axis_index("s")
    wid = sid * NC + cid

    zero = jnp.zeros((L,), jnp.float32)

    for r in range(K):
        for c in range(D // L):
            rows_v0[r, pl.ds(c * L, L)] = zero

    def _zdeg(q, carry):
        deg_v[pl.ds(q * L, L)] = zero
        return carry

    lax.fori_loop(0, NP // L, _zdeg, 0)

    for b in range(NCP):
        r0 = sid * RPT + b * CPB
        pltpu.sync_copy(rows_v0, acc_sh.at[pl.ds(r0, CPB)])

    plsc.subcore_barrier()

    bufs = ((src_v0, dst_v0, w_v0, rows_v0, sem0, ssem0),
            (src_v1, dst_v1, w_v1, rows_v1, sem1, ssem1))

    def _fetch_idx(g, b):
        src_v, dst_v, w_v, _, _, _ = bufs[b]
        off = pl.multiple_of(wid * EPW + g * K, K)
        pltpu.sync_copy(src_hbm.at[pl.ds(off, K)], src_v)
        pltpu.sync_copy(dst_hbm.at[pl.ds(off, K)], dst_v)
        pltpu.sync_copy(w_hbm.at[pl.ds(off, K)], w_v)

    def _start_gather(b):
        _, dst_v, _, rows_v, sem, _ = bufs[b]
        return pltpu.async_copy(x_hbm.at[dst_v], rows_v, sem)

    def _process(g, b, prefetch_next, wait_prev_scatter):
        src_v, dst_v, w_v, rows_v, sem, ssem = bufs[b]
        if prefetch_next:
            _fetch_idx(g + 1, 1 - b)

        for q in range(K // L):
            idxv = src_v[pl.ds(q * L, L)]
            wg0 = w_v[pl.ds(q * L, L)]
            plsc.addupdate_scatter(deg_v, [idxv], wg0)

        if prefetch_next:
            if wait_prev_scatter:
                _, _, _, prows, _, pssem = bufs[1 - b]
                pltpu.make_async_copy(prows, acc_sh.at[dst_v], pssem).wait()
            _start_gather(1 - b)

        pltpu.make_async_copy(x_hbm.at[dst_v], rows_v, sem).wait()

        @plsc.parallel_loop(0, K // L, step=1)
        def _scale(q):
            wg = w_v[pl.ds(q * L, L)]
            for t in range(L):
                r = q * L + t
                wv = jnp.zeros((L,), jnp.float32) + wg[t]
                for c in range(D // L):
                    rows_v[r, pl.ds(c * L, L)] = rows_v[r, pl.ds(c * L, L)] * wv

        pltpu.async_copy(rows_v, acc_sh.at[src_v], ssem, add=True)

    _fetch_idx(0, 0)
    _start_gather(0)
    _process(0, 0, True, False)

    def _pair(p, carry):
        g = 2 * p + 1
        _process(g, 1, True, True)
        _process(g + 1, 0, True, True)
        return carry

    lax.fori_loop(0, (NCHUNK - 3) // 2, _pair, 0)
    _process(NCHUNK - 2, 1, True, True)
    _process(NCHUNK - 1, 0, False, False)

    pltpu.make_async_copy(rows_v1, acc_sh.at[dst_v1], ssem1).wait()
    pltpu.make_async_copy(rows_v0, acc_sh.at[dst_v0], ssem0).wait()

    plsc.subcore_barrier()

    for b in range(NCP):
        r0 = sid * RPT + b * CPB
        pltpu.sync_copy(acc_sh.at[pl.ds(r0, CPB)], rows_v0)
        pltpu.sync_copy(rows_v0, acc_out.at[pl.ds(cid * NP + r0, CPB)])

    pltpu.sync_copy(deg_v, deg_out.at[pl.ds(wid * NP, NP)])


def _finalize_body(acc_ref, deg_ref, gw_ref, gb_ref, gms_ref, out_ref):
    a = acc_ref[0:N, :] + acc_ref[NP:NP + N, :]
    ones = jnp.ones((NW, 1), jnp.float32)
    d_col = lax.dot_general(deg_ref[...], ones, (((0,), (0,)), ((), ())),
                            preferred_element_type=jnp.float32)
    d = d_col[0:N, :]
    d = jnp.where(d < 0.5, d + 1.0, d)
    x = a * (1.0 / d)
    mean = jnp.sum(x, axis=0, keepdims=True) * (1.0 / N)
    centered = x - mean * gms_ref[...]
    var = jnp.sum(centered * centered, axis=0, keepdims=True) * (1.0 / N)
    inv_std = lax.rsqrt(var + 1e-6)
    out_ref[...] = gw_ref[...] * centered * inv_std + gb_ref[...]


_finalize = pl.pallas_call(
    _finalize_body,
    out_shape=jax.ShapeDtypeStruct((N, D), jnp.float32),
)


@jax.jit
def kernel(x_, edge_index, edge_weight, gn_weight, gn_bias, gn_mean_scale):
    src = edge_index[0]
    dst = edge_index[1]
    acc2, deg2 = _sc_accumulate(src, dst, edge_weight, x_)
    return _finalize(
        acc2,
        deg2.reshape(NW, NP),
        gn_weight.reshape(1, D),
        gn_bias.reshape(1, D),
        gn_mean_scale.reshape(1, D),
    )


# speedup vs baseline: 1.3645x; 1.3645x over previous
import functools

import jax
import jax.numpy as jnp
from jax import lax
from jax.experimental import pallas as pl
from jax.experimental.pallas import tpu as pltpu
from jax.experimental.pallas import tpu_sc as plsc

N = 10000
NP = 10240
E = 320000
D = 128
L = 16
NC = 2
NS = 16
NW = NC * NS
EPW = E // NW
K = 80
NCHUNK = EPW // K
NCHT = E // K
EROW = 256
RPT = NP // NS
CPB = K
NCP = RPT // CPB

_mesh = plsc.VectorSubcoreMesh(
    core_axis_name="c", subcore_axis_name="s", num_cores=NC, num_subcores=NS
)


@functools.partial(
    pl.kernel,
    out_type=[
        jax.ShapeDtypeStruct((NC * NP, D), jnp.float32),
        jax.ShapeDtypeStruct((NW * NP,), jnp.float32),
    ],
    mesh=_mesh,
    compiler_params=pltpu.CompilerParams(needs_layout_passes=False),
    scratch_types=[
        pltpu.VMEM((EROW,), jnp.int32),
        pltpu.VMEM((EROW,), jnp.int32),
        pltpu.VMEM((K,), jnp.int32),
        pltpu.VMEM((K,), jnp.int32),
        pltpu.VMEM((K, D), jnp.float32),
        pltpu.VMEM((K, D), jnp.float32),
        pltpu.VMEM((NP,), jnp.float32),
        pltpu.VMEM_SHARED((NP, D), jnp.float32),
        pltpu.SemaphoreType.DMA,
        pltpu.SemaphoreType.DMA,
        pltpu.SemaphoreType.DMA,
        pltpu.SemaphoreType.DMA,
        pltpu.SemaphoreType.DMA,
        pltpu.SemaphoreType.DMA,
    ],
)
def _sc_accumulate(
    epk_hbm, x_hbm,
    acc_out, deg_out,
    ebuf0, ebuf1, src_v0, src_v1,
    rows_v0, rows_v1, deg_v, acc_sh,
    sem0, sem1, ssem0, ssem1, isem0, isem1,
):
    cid = lax.axis_index("c")
    sid = lax.axis_index("s")
    wid = sid * NC + cid

    zero = jnp.zeros((L,), jnp.float32)

    for r in range(K):
        for c in range(D // L):
            rows_v0[r, pl.ds(c * L, L)] = zero

    def _zdeg(q, carry):
        deg_v[pl.ds(q * L, L)] = zero
        return carry

    lax.fori_loop(0, NP // L, _zdeg, 0)

    for b in range(NCP):
        r0 = sid * RPT + b * CPB
        pltpu.sync_copy(rows_v0, acc_sh.at[pl.ds(r0, CPB)])

    plsc.subcore_barrier()

    bufs = ((ebuf0, src_v0, rows_v0, sem0, ssem0, isem0),
            (ebuf1, src_v1, rows_v1, sem1, ssem1, isem1))

    def _row_ds(g):
        cg = wid * NCHUNK + g
        return pl.ds(pl.multiple_of(cg * EROW, EROW), EROW)

    def _fetch_idx(g, b):
        pltpu.async_copy(epk_hbm.at[_row_ds(g)], bufs[b][0], bufs[b][5])

    def _start_gather(b):
        ebuf, _, rows_v, sem, _, _ = bufs[b]
        pltpu.async_copy(x_hbm.at[ebuf.at[pl.ds(K, K)]], rows_v, sem)

    def _process(g, b, prefetch_next, wait_prev_scatter):
        ebuf, src_v, rows_v, sem, ssem, isem = bufs[b]
        if prefetch_next:
            _fetch_idx(g + 1, 1 - b)

        for q in range(K // L):
            idxv = ebuf[pl.ds(q * L, L)]
            wg0 = plsc.bitcast(ebuf[pl.ds(2 * K + q * L, L)], jnp.float32)
            plsc.addupdate_scatter(deg_v, [idxv], wg0)

        if prefetch_next:
            pebuf, _, prows, _, pssem, pisem = bufs[1 - b]
            if wait_prev_scatter:
                pltpu.make_async_copy(prows, acc_sh.at[src_v], pssem).wait()
            pltpu.make_async_copy(epk_hbm.at[_row_ds(g + 1)], pebuf,
                                  pisem).wait()
            _start_gather(1 - b)

        pltpu.make_async_copy(x_hbm.at[ebuf.at[pl.ds(K, K)]], rows_v,
                              sem).wait()

        @plsc.parallel_loop(0, K // L, step=1)
        def _scale(q):
            src_v[pl.ds(q * L, L)] = ebuf[pl.ds(q * L, L)]
            wg = plsc.bitcast(ebuf[pl.ds(2 * K + q * L, L)], jnp.float32)
            for t in range(L):
                r = q * L + t
                wv = jnp.zeros((L,), jnp.float32) + wg[t]
                for c in range(D // L):
                    rows_v[r, pl.ds(c * L, L)] = rows_v[r, pl.ds(c * L, L)] * wv

        pltpu.async_copy(rows_v, acc_sh.at[src_v], ssem, add=True)

    _fetch_idx(0, 0)
    pltpu.make_async_copy(epk_hbm.at[_row_ds(0)], ebuf0, isem0).wait()
    _start_gather(0)
    _process(0, 0, True, False)

    def _pair(p, carry):
        g = 2 * p + 1
        _process(g, 1, True, True)
        _process(g + 1, 0, True, True)
        return carry

    lax.fori_loop(0, (NCHUNK - 3) // 2, _pair, 0)
    _process(NCHUNK - 2, 1, True, True)
    _process(NCHUNK - 1, 0, False, False)

    pltpu.make_async_copy(rows_v1, acc_sh.at[src_v1], ssem1).wait()
    pltpu.make_async_copy(rows_v0, acc_sh.at[src_v0], ssem0).wait()

    plsc.subcore_barrier()

    for b in range(NCP):
        r0 = sid * RPT + b * CPB
        pltpu.sync_copy(acc_sh.at[pl.ds(r0, CPB)], rows_v0)
        pltpu.sync_copy(rows_v0, acc_out.at[pl.ds(cid * NP + r0, CPB)])

    pltpu.sync_copy(deg_v, deg_out.at[pl.ds(wid * NP, NP)])


def _finalize_body(acc_ref, deg_ref, gw_ref, gb_ref, gms_ref, out_ref):
    a = acc_ref[0:N, :] + acc_ref[NP:NP + N, :]
    ones = jnp.ones((NW, 1), jnp.float32)
    d_col = lax.dot_general(deg_ref[...], ones, (((0,), (0,)), ((), ())),
                            preferred_element_type=jnp.float32)
    d = d_col[0:N, :]
    d = jnp.where(d < 0.5, d + 1.0, d)
    x = a * (1.0 / d)
    mean = jnp.sum(x, axis=0, keepdims=True) * (1.0 / N)
    centered = x - mean * gms_ref[...]
    var = jnp.sum(centered * centered, axis=0, keepdims=True) * (1.0 / N)
    inv_std = lax.rsqrt(var + 1e-6)
    out_ref[...] = gw_ref[...] * centered * inv_std + gb_ref[...]


_finalize = pl.pallas_call(
    _finalize_body,
    out_shape=jax.ShapeDtypeStruct((N, D), jnp.float32),
)


@jax.jit
def kernel(x_, edge_index, edge_weight, gn_weight, gn_bias, gn_mean_scale):
    srcM = edge_index[0].reshape(NCHT, K)
    dstM = edge_index[1].reshape(NCHT, K)
    wM = lax.bitcast_convert_type(edge_weight, jnp.int32).reshape(NCHT, K)
    pad = jnp.zeros((NCHT, EROW - 3 * K), jnp.int32)
    epk = jnp.concatenate([srcM, dstM, wM, pad], axis=1).reshape(-1)
    acc2, deg2 = _sc_accumulate(epk, x_)
    return _finalize(
        acc2,
        deg2.reshape(NW, NP),
        gn_weight.reshape(1, D),
        gn_bias.reshape(1, D),
        gn_mean_scale.reshape(1, D),
    )
